# baseline (speedup 1.0000x reference)
import dataclasses
import functools

import numpy as np
import jax
import jax.numpy as jnp
from jax import lax
from jax.experimental import pallas as pl
from jax.experimental.pallas import tpu as pltpu
from jax.experimental.pallas import tpu_sc as plsc

NC = 2
NS = 16
NW = NC * NS
LANES = 16
CHUNK = 1024
NBUF = 3
NBUF_T = 8


def kernel(x, input_embedding_table):
    B, H = x.shape
    V, D = input_embedding_table.shape
    n = B * H
    assert n % (NW * CHUNK) == 0 and D == 2 * LANES
    per_w = n // NW
    n_chunks = per_w // CHUNK
    scale = float(np.sqrt(np.float32(D)))

    tcol_full = V // 128
    tail_w = V - tcol_full * 128
    main = (tcol_full // NW) // NBUF_T * NBUF_T

    table_t = input_embedding_table.T
    idx = x.reshape(n)
    mesh = plsc.VectorSubcoreMesh(core_axis_name="c", subcore_axis_name="s")

    @functools.partial(
        pl.kernel,
        mesh=mesh,
        compiler_params=dataclasses.replace(
            pltpu.CompilerParams(use_tc_tiling_on_sc=True),
            needs_layout_passes=False,
        ),
        out_type=jax.ShapeDtypeStruct((V * D,), jnp.float32),
        scratch_types=(
            [pltpu.VMEM((4, 8, 128), jnp.float32)] * NBUF_T
            + [pltpu.VMEM((128 * 32,), jnp.float32)] * NBUF_T
            + [
                pltpu.VMEM((4, 8, tail_w), jnp.float32),
                pltpu.VMEM((tail_w * 32,), jnp.float32),
                pltpu.SemaphoreType.DMA((NBUF_T,)),
                pltpu.SemaphoreType.DMA((NBUF_T,)),
            ]
        ),
    )
    def sc_transpose(tbl_hbm, out_hbm, *scr):
        in_bufs = scr[0:NBUF_T]
        out_bufs = scr[NBUF_T : 2 * NBUF_T]
        tin_v, tout_v, isem, osem = scr[2 * NBUF_T :]
        wid = lax.axis_index("s") * NC + lax.axis_index("c")
        iota = lax.iota(jnp.int32, LANES)
        dt_lo = iota // 8
        s_ix = iota % 8

        def col_in_start(c, b):
            for dt in range(4):
                pltpu.async_copy(
                    tbl_hbm.at[pl.ds(dt * 8, 8), pl.ds(c * 128, 128)],
                    in_bufs[b].at[dt],
                    isem.at[b],
                )

        def col_in_wait(b):
            for dt in range(4):
                pltpu.make_async_copy(
                    tbl_hbm.at[pl.ds(dt * 8, 8), pl.ds(0, 128)],
                    in_bufs[b].at[dt],
                    isem.at[b],
                ).wait()

        def out_wait(b):
            pltpu.make_async_copy(
                out_bufs[b],
                out_hbm.at[pl.ds(0, 128 * 32)],
                osem.at[b],
            ).wait()

        def transpose_into(inref4, outref, width, base):
            @plsc.parallel_loop(0, width, unroll=8)
            def _(l):
                lv = jnp.full((LANES,), 0, jnp.int32) + l
                g0 = plsc.load_gather(inref4, [dt_lo, s_ix, lv])
                g1 = plsc.load_gather(inref4, [dt_lo + 2, s_ix, lv])
                outref[pl.ds(base + l * 32, LANES)] = g0 * scale
                outref[pl.ds(base + l * 32 + LANES, LANES)] = g1 * scale

        for b0 in range(NBUF_T):
            col_in_start(wid + b0 * NW, b0)

        @pl.loop(0, main, step=NBUF_T)
        def _(j):
            for b in range(NBUF_T):
                jj = j + b
                c = wid + jj * NW
                col_in_wait(b)

                @pl.when(jj >= NBUF_T)
                def _():
                    out_wait(b)

                transpose_into(in_bufs[b], out_bufs[b], 128, 0)

                pltpu.async_copy(
                    out_bufs[b],
                    out_hbm.at[pl.ds(c * 4096, 4096)],
                    osem.at[b],
                )

                @pl.when(jj + NBUF_T < main)
                def _():
                    col_in_start(wid + (jj + NBUF_T) * NW, b)

        for b in range(NBUF_T):
            out_wait(b)

        nrest = tcol_full - main * NW
        rest_per_w = nrest // NW
        extra = nrest - rest_per_w * NW
        for r in range(rest_per_w):
            c = (main + r) * NW + wid
            for dt in range(4):
                pltpu.sync_copy(
                    tbl_hbm.at[pl.ds(dt * 8, 8), pl.ds(c * 128, 128)],
                    in_bufs[0].at[dt],
                )
            transpose_into(in_bufs[0], out_bufs[0], 128, 0)
            pltpu.sync_copy(out_bufs[0], out_hbm.at[pl.ds(c * 4096, 4096)])

        if extra:
            @pl.when(wid < extra)
            def _():
                c = (main + rest_per_w) * NW + wid
                for dt in range(4):
                    pltpu.sync_copy(
                        tbl_hbm.at[pl.ds(dt * 8, 8), pl.ds(c * 128, 128)],
                        in_bufs[0].at[dt],
                    )
                transpose_into(in_bufs[0], out_bufs[0], 128, 0)
                pltpu.sync_copy(
                    out_bufs[0], out_hbm.at[pl.ds(c * 4096, 4096)]
                )

        @pl.when(wid == NW - 1)
        def _():
            for dt in range(4):
                pltpu.sync_copy(
                    tbl_hbm.at[pl.ds(dt * 8, 8), pl.ds(tcol_full * 128, tail_w)],
                    tin_v.at[dt],
                )
            transpose_into(tin_v, tout_v, tail_w, 0)
            pltpu.sync_copy(
                tout_v, out_hbm.at[pl.ds(tcol_full * 4096, tail_w * 32)]
            )

    tbl_lin = sc_transpose(table_t)
    tbl2d = tbl_lin.reshape(V, D)

    @functools.partial(
        pl.kernel,
        mesh=mesh,
        compiler_params=pltpu.CompilerParams(use_tc_tiling_on_sc=False),
        out_type=jax.ShapeDtypeStruct((n, D), jnp.float32),
        scratch_types=[
            pltpu.VMEM((per_w,), jnp.int32),
            pltpu.VMEM((NBUF, CHUNK, D), jnp.float32),
            pltpu.SemaphoreType.DMA((NBUF,)),
            pltpu.SemaphoreType.DMA((NBUF,)),
        ],
    )
    def sc_gather(table_hbm, idx_hbm, out_hbm, idx_v, rows_v, gsem, osem):
        wid = lax.axis_index("s") * NC + lax.axis_index("c")
        base = wid * per_w
        pltpu.sync_copy(idx_hbm.at[pl.ds(base, per_w)], idx_v)

        def start_gather(ci):
            b = ci % NBUF
            return pltpu.async_copy(
                table_hbm.at[idx_v.at[pl.ds(ci * CHUNK, CHUNK)]],
                rows_v.at[b],
                gsem.at[b],
            )

        def start_out(ci):
            b = ci % NBUF
            return pltpu.async_copy(
                rows_v.at[b],
                out_hbm.at[pl.ds(base + ci * CHUNK, CHUNK)],
                osem.at[b],
            )

        gathers = {0: start_gather(0), 1: start_gather(1)}
        outs = {}
        for ci in range(n_chunks):
            nxt = ci + 2
            if nxt < n_chunks:
                if nxt - NBUF in outs:
                    outs[nxt - NBUF].wait()
                gathers[nxt] = start_gather(nxt)
            gathers[ci].wait()
            outs[ci] = start_out(ci)
        for ci in range(max(0, n_chunks - NBUF), n_chunks):
            if ci in outs:
                outs[ci].wait()

    out = sc_gather(tbl2d, idx)
    return out.reshape(B, H, D)


# speedup vs baseline: 1.4629x; 1.4629x over previous
import dataclasses
import functools

import numpy as np
import jax
import jax.numpy as jnp
from jax import lax
from jax.experimental import pallas as pl
from jax.experimental.pallas import tpu as pltpu
from jax.experimental.pallas import tpu_sc as plsc

NC = 2
NS = 16
NW = NC * NS
LANES = 16
CHUNK = 1024
NBUF = 3
NBUF_T = 8


def kernel(x, input_embedding_table):
    B, H = x.shape
    V, D = input_embedding_table.shape
    n = B * H
    assert n % (NW * CHUNK) == 0 and D == 2 * LANES
    per_w = n // NW
    n_chunks = per_w // CHUNK
    scale = float(np.sqrt(np.float32(D)))

    tcol_full = V // 128
    tail_w = V - tcol_full * 128
    main = (tcol_full // NW) // NBUF_T * NBUF_T

    table_t = input_embedding_table.T
    idx = x.reshape(n)
    mesh = plsc.VectorSubcoreMesh(core_axis_name="c", subcore_axis_name="s")

    @functools.partial(
        pl.kernel,
        mesh=mesh,
        compiler_params=dataclasses.replace(
            pltpu.CompilerParams(use_tc_tiling_on_sc=True),
            needs_layout_passes=False,
        ),
        out_type=jax.ShapeDtypeStruct((V * D,), jnp.float32),
        scratch_types=(
            [pltpu.VMEM((4, 8, 128), jnp.float32)] * NBUF_T
            + [pltpu.VMEM((128 * 32,), jnp.float32)] * NBUF_T
            + [
                pltpu.VMEM((128 * 33,), jnp.float32),
                pltpu.VMEM((4, 8, tail_w), jnp.float32),
                pltpu.VMEM((tail_w * 32,), jnp.float32),
                pltpu.SemaphoreType.DMA((NBUF_T,)),
                pltpu.SemaphoreType.DMA((NBUF_T,)),
            ]
        ),
    )
    def sc_transpose(tbl_hbm, out_hbm, *scr):
        in_bufs = scr[0:NBUF_T]
        out_bufs = scr[NBUF_T : 2 * NBUF_T]
        pad_v, tin_v, tout_v, isem, osem = scr[2 * NBUF_T :]
        wid = lax.axis_index("s") * NC + lax.axis_index("c")
        iota = lax.iota(jnp.int32, LANES)

        def col_in_start(c, b):
            for dt in range(4):
                pltpu.async_copy(
                    tbl_hbm.at[pl.ds(dt * 8, 8), pl.ds(c * 128, 128)],
                    in_bufs[b].at[dt],
                    isem.at[b],
                )

        def col_in_wait(b):
            for dt in range(4):
                pltpu.make_async_copy(
                    tbl_hbm.at[pl.ds(dt * 8, 8), pl.ds(0, 128)],
                    in_bufs[b].at[dt],
                    isem.at[b],
                ).wait()

        def out_wait(b):
            pltpu.make_async_copy(
                out_bufs[b],
                out_hbm.at[pl.ds(0, 128 * 32)],
                osem.at[b],
            ).wait()

        iota33 = iota * 33

        def transpose_into(inref, outref, width, pad_buf):
            @plsc.parallel_loop(0, width // 16, unroll=1)
            def _(l0):
                for d in range(32):
                    dt, sl = d // 8, d % 8
                    v = inref[dt, sl, pl.ds(l0 * 16, 16)]
                    plsc.store_scatter(
                        pad_buf, [iota33 + (l0 * 528 + d)], v * scale
                    )

            @plsc.parallel_loop(0, width, unroll=4)
            def _(l):
                g0 = plsc.load_gather(pad_buf, [iota + l * 33])
                g1 = plsc.load_gather(pad_buf, [iota + (l * 33 + 16)])
                outref[pl.ds(l * 32, LANES)] = g0
                outref[pl.ds(l * 32 + LANES, LANES)] = g1

        for b0 in range(NBUF_T):
            col_in_start(wid + b0 * NW, b0)

        @pl.loop(0, main, step=NBUF_T)
        def _(j):
            for b in range(NBUF_T):
                jj = j + b
                c = wid + jj * NW
                col_in_wait(b)

                @pl.when(jj >= NBUF_T)
                def _():
                    out_wait(b)

                transpose_into(in_bufs[b], out_bufs[b], 128, pad_v)

                pltpu.async_copy(
                    out_bufs[b],
                    out_hbm.at[pl.ds(c * 4096, 4096)],
                    osem.at[b],
                )

                @pl.when(jj + NBUF_T < main)
                def _():
                    col_in_start(wid + (jj + NBUF_T) * NW, b)

        for b in range(NBUF_T):
            out_wait(b)

        nrest = tcol_full - main * NW
        rest_per_w = nrest // NW
        extra = nrest - rest_per_w * NW
        for r in range(rest_per_w):
            c = (main + r) * NW + wid
            for dt in range(4):
                pltpu.sync_copy(
                    tbl_hbm.at[pl.ds(dt * 8, 8), pl.ds(c * 128, 128)],
                    in_bufs[0].at[dt],
                )
            transpose_into(in_bufs[0], out_bufs[0], 128, pad_v)
            pltpu.sync_copy(out_bufs[0], out_hbm.at[pl.ds(c * 4096, 4096)])

        if extra:
            @pl.when(wid < extra)
            def _():
                c = (main + rest_per_w) * NW + wid
                for dt in range(4):
                    pltpu.sync_copy(
                        tbl_hbm.at[pl.ds(dt * 8, 8), pl.ds(c * 128, 128)],
                        in_bufs[0].at[dt],
                    )
                transpose_into(in_bufs[0], out_bufs[0], 128, pad_v)
                pltpu.sync_copy(
                    out_bufs[0], out_hbm.at[pl.ds(c * 4096, 4096)]
                )

        @pl.when(wid == NW - 1)
        def _():
            for dt in range(4):
                pltpu.sync_copy(
                    tbl_hbm.at[pl.ds(dt * 8, 8), pl.ds(tcol_full * 128, tail_w)],
                    tin_v.at[dt],
                )
            transpose_into(tin_v, tout_v, tail_w, pad_v)
            pltpu.sync_copy(
                tout_v, out_hbm.at[pl.ds(tcol_full * 4096, tail_w * 32)]
            )

    tbl_lin = sc_transpose(table_t)
    tbl2d = tbl_lin.reshape(V, D)

    @functools.partial(
        pl.kernel,
        mesh=mesh,
        compiler_params=pltpu.CompilerParams(use_tc_tiling_on_sc=False),
        out_type=jax.ShapeDtypeStruct((n, D), jnp.float32),
        scratch_types=[
            pltpu.VMEM((per_w,), jnp.int32),
            pltpu.VMEM((NBUF, CHUNK, D), jnp.float32),
            pltpu.SemaphoreType.DMA((NBUF,)),
            pltpu.SemaphoreType.DMA((NBUF,)),
        ],
    )
    def sc_gather(table_hbm, idx_hbm, out_hbm, idx_v, rows_v, gsem, osem):
        wid = lax.axis_index("s") * NC + lax.axis_index("c")
        base = wid * per_w
        pltpu.sync_copy(idx_hbm.at[pl.ds(base, per_w)], idx_v)

        def start_gather(ci):
            b = ci % NBUF
            return pltpu.async_copy(
                table_hbm.at[idx_v.at[pl.ds(ci * CHUNK, CHUNK)]],
                rows_v.at[b],
                gsem.at[b],
            )

        def start_out(ci):
            b = ci % NBUF
            return pltpu.async_copy(
                rows_v.at[b],
                out_hbm.at[pl.ds(base + ci * CHUNK, CHUNK)],
                osem.at[b],
            )

        gathers = {0: start_gather(0), 1: start_gather(1)}
        outs = {}
        for ci in range(n_chunks):
            nxt = ci + 2
            if nxt < n_chunks:
                if nxt - NBUF in outs:
                    outs[nxt - NBUF].wait()
                gathers[nxt] = start_gather(nxt)
            gathers[ci].wait()
            outs[ci] = start_out(ci)
        for ci in range(max(0, n_chunks - NBUF), n_chunks):
            if ci in outs:
                outs[ci].wait()

    out = sc_gather(tbl2d, idx)
    return out.reshape(B, H, D)


# speedup vs baseline: 2.1461x; 1.4670x over previous
import dataclasses
import functools

import numpy as np
import jax
import jax.numpy as jnp
from jax import lax
from jax.experimental import pallas as pl
from jax.experimental.pallas import tpu as pltpu
from jax.experimental.pallas import tpu_sc as plsc

NC = 2
NS = 16
NW = NC * NS
LANES = 16
CHUNK = 1024
NBUF = 3
NBUF_T = 8


def kernel(x, input_embedding_table):
    B, H = x.shape
    V, D = input_embedding_table.shape
    n = B * H
    assert n % (NW * CHUNK) == 0 and D == 2 * LANES
    per_w = n // NW
    n_chunks = per_w // CHUNK
    scale = float(np.sqrt(np.float32(D)))

    tcol_full = V // 128
    tail_w = V - tcol_full * 128
    main = (tcol_full // NW) // NBUF_T * NBUF_T

    table_t = input_embedding_table.T
    idx = x.reshape(n)
    mesh = plsc.VectorSubcoreMesh(core_axis_name="c", subcore_axis_name="s")

    @functools.partial(
        pl.kernel,
        mesh=mesh,
        compiler_params=dataclasses.replace(
            pltpu.CompilerParams(use_tc_tiling_on_sc=True),
            needs_layout_passes=False,
        ),
        out_type=jax.ShapeDtypeStruct((V * D,), jnp.float32),
        scratch_types=(
            [pltpu.VMEM((4, 8, 128), jnp.float32)] * NBUF_T
            + [pltpu.VMEM((128 * 32,), jnp.float32)] * NBUF_T
            + [
                pltpu.VMEM((128 * 33,), jnp.float32),
                pltpu.VMEM((4, 8, tail_w), jnp.float32),
                pltpu.VMEM((tail_w * 32,), jnp.float32),
                pltpu.SemaphoreType.DMA((NBUF_T,)),
                pltpu.SemaphoreType.DMA((NBUF_T,)),
            ]
        ),
    )
    def sc_transpose(tbl_hbm, out_hbm, *scr):
        in_bufs = scr[0:NBUF_T]
        out_bufs = scr[NBUF_T : 2 * NBUF_T]
        pad_v, tin_v, tout_v, isem, osem = scr[2 * NBUF_T :]
        wid = lax.axis_index("s") * NC + lax.axis_index("c")
        iota = lax.iota(jnp.int32, LANES)

        def col_in_start(c, b):
            for dt in range(4):
                pltpu.async_copy(
                    tbl_hbm.at[pl.ds(dt * 8, 8), pl.ds(c * 128, 128)],
                    in_bufs[b].at[dt],
                    isem.at[b],
                )

        def col_in_wait(b):
            for dt in range(4):
                pltpu.make_async_copy(
                    tbl_hbm.at[pl.ds(dt * 8, 8), pl.ds(0, 128)],
                    in_bufs[b].at[dt],
                    isem.at[b],
                ).wait()

        def out_wait(b):
            pltpu.make_async_copy(
                out_bufs[b],
                out_hbm.at[pl.ds(0, 128 * 32)],
                osem.at[b],
            ).wait()

        iota33 = iota * 33

        def transpose_into(inref, outref, width, pad_buf):
            @plsc.parallel_loop(0, width // 16, unroll=1)
            def _(l0):
                for d in range(32):
                    dt, sl = d // 8, d % 8
                    v = inref[dt, sl, pl.ds(l0 * 16, 16)]
                    plsc.store_scatter(
                        pad_buf, [iota33 + (l0 * 528 + d)], v * scale
                    )

            @plsc.parallel_loop(0, width, unroll=4)
            def _(l):
                g0 = plsc.load_gather(pad_buf, [iota + l * 33])
                g1 = plsc.load_gather(pad_buf, [iota + (l * 33 + 16)])
                outref[pl.ds(l * 32, LANES)] = g0
                outref[pl.ds(l * 32 + LANES, LANES)] = g1

        for b0 in range(NBUF_T):
            col_in_start(wid + b0 * NW, b0)

        @pl.loop(0, main, step=NBUF_T)
        def _(j):
            for b in range(NBUF_T):
                jj = j + b
                c = wid + jj * NW
                col_in_wait(b)

                @pl.when(jj >= NBUF_T)
                def _():
                    out_wait(b)

                transpose_into(in_bufs[b], out_bufs[b], 128, pad_v)

                pltpu.async_copy(
                    out_bufs[b],
                    out_hbm.at[pl.ds(c * 4096, 4096)],
                    osem.at[b],
                )

                @pl.when(jj + NBUF_T < main)
                def _():
                    col_in_start(wid + (jj + NBUF_T) * NW, b)

        for b in range(NBUF_T):
            out_wait(b)

        nrest = tcol_full - main * NW
        rest_per_w = nrest // NW
        extra = nrest - rest_per_w * NW
        for r in range(rest_per_w):
            c = (main + r) * NW + wid
            for dt in range(4):
                pltpu.sync_copy(
                    tbl_hbm.at[pl.ds(dt * 8, 8), pl.ds(c * 128, 128)],
                    in_bufs[0].at[dt],
                )
            transpose_into(in_bufs[0], out_bufs[0], 128, pad_v)
            pltpu.sync_copy(out_bufs[0], out_hbm.at[pl.ds(c * 4096, 4096)])

        if extra:
            @pl.when(wid < extra)
            def _():
                c = (main + rest_per_w) * NW + wid
                for dt in range(4):
                    pltpu.sync_copy(
                        tbl_hbm.at[pl.ds(dt * 8, 8), pl.ds(c * 128, 128)],
                        in_bufs[0].at[dt],
                    )
                transpose_into(in_bufs[0], out_bufs[0], 128, pad_v)
                pltpu.sync_copy(
                    out_bufs[0], out_hbm.at[pl.ds(c * 4096, 4096)]
                )

        @pl.when(wid == NW - 1)
        def _():
            for dt in range(4):
                pltpu.sync_copy(
                    tbl_hbm.at[pl.ds(dt * 8, 8), pl.ds(tcol_full * 128, tail_w)],
                    tin_v.at[dt],
                )
            transpose_into(tin_v, tout_v, tail_w, pad_v)
            pltpu.sync_copy(
                tout_v, out_hbm.at[pl.ds(tcol_full * 4096, tail_w * 32)]
            )

    tbl_lin = sc_transpose(table_t)
    tbl2d = tbl_lin.reshape(V, D)

    HG = 5
    PB = 128
    BLK = PB * H
    n_blk = per_w // BLK
    PADW = 129
    PSTRIDE_DT = 8 * PADW
    PSTRIDE_HL = 4 * PSTRIDE_DT

    @functools.partial(
        pl.kernel,
        mesh=mesh,
        compiler_params=dataclasses.replace(
            pltpu.CompilerParams(use_tc_tiling_on_sc=False),
            needs_layout_passes=False,
        ),
        out_type=jax.ShapeDtypeStruct((H, 4, B // 128, 8, 128), jnp.float32),
        scratch_types=[
            pltpu.VMEM((per_w,), jnp.int32),
            pltpu.VMEM((BLK, D), jnp.float32),
            pltpu.VMEM((HG * PSTRIDE_HL,), jnp.float32),
            pltpu.VMEM((1, 1, 1, 8, 128), jnp.float32),
            pltpu.VMEM((1, 1, 1, 8, 128), jnp.float32),
            pltpu.SemaphoreType.DMA,
            pltpu.SemaphoreType.DMA((2,)),
        ],
    )
    def sc_gather(
        table_hbm, idx_hbm, out_hbm, idx_v, rows_v, piece_v, db0, db1, gsem, osem
    ):
        wid = lax.axis_index("s") * NC + lax.axis_index("c")
        base = wid * per_w
        iota = lax.iota(jnp.int32, LANES)
        pre16 = (iota // 8) * PSTRIDE_DT + (iota % 8) * PADW
        dbufs = (db0, db1)

        pltpu.sync_copy(idx_hbm.at[pl.ds(base, per_w)], idx_v)

        for blk in range(n_blk):
            pltpu.async_copy(
                table_hbm.at[idx_v.at[pl.ds(blk * BLK, BLK)]],
                rows_v,
                gsem,
            ).wait()

            @pl.loop(0, 4)
            def _(hg):
                h0 = hg * HG

                @plsc.parallel_loop(0, PB, unroll=2)
                def _(bi):
                    rb = bi * H
                    for hl in range(HG):
                        r = rb + h0 + hl
                        loc = hl * PSTRIDE_HL + bi
                        v0 = rows_v[r, pl.ds(0, LANES)]
                        v1 = rows_v[r, pl.ds(LANES, LANES)]
                        plsc.store_scatter(piece_v, [pre16 + loc], v0)
                        plsc.store_scatter(
                            piece_v, [pre16 + (loc + 2 * PSTRIDE_DT)], v1
                        )

                @pl.loop(0, 20, step=2)
                def _(pp):
                    for sub in range(2):
                        pc = pp + sub
                        buf = dbufs[sub]
                        hl = pc // 4
                        dt = pc % 4
                        pbase = hl * PSTRIDE_HL + dt * PSTRIDE_DT

                        @pl.when(pc >= 2)
                        def _():
                            pltpu.make_async_copy(
                                buf,
                                out_hbm.at[
                                    pl.ds(0, 1), pl.ds(0, 1), pl.ds(0, 1)
                                ],
                                osem.at[sub],
                            ).wait()

                        @plsc.parallel_loop(0, 8, unroll=2)
                        def _(di):
                            off = pbase + di * PADW
                            for g in range(8):
                                buf[0, 0, 0, di, pl.ds(g * 16, 16)] = (
                                    piece_v[pl.ds(off + g * 16, 16)]
                                )

                        pltpu.async_copy(
                            buf,
                            out_hbm.at[
                                pl.ds(h0 + hl, 1),
                                pl.ds(dt, 1),
                                pl.ds(wid * n_blk + blk, 1),
                            ],
                            osem.at[sub],
                        )

                for sub in range(2):
                    pltpu.make_async_copy(
                        dbufs[sub],
                        out_hbm.at[pl.ds(0, 1), pl.ds(0, 1), pl.ds(0, 1)],
                        osem.at[sub],
                    ).wait()

    out5 = sc_gather(tbl2d, idx)
    return out5.transpose(2, 4, 0, 1, 3).reshape(B, H, D)


# speedup vs baseline: 2.1879x; 1.0195x over previous
import dataclasses
import functools

import numpy as np
import jax
import jax.numpy as jnp
from jax import lax
from jax.experimental import pallas as pl
from jax.experimental.pallas import tpu as pltpu
from jax.experimental.pallas import tpu_sc as plsc

NC = 2
NS = 16
NW = NC * NS
LANES = 16
NBUF_T = 8
PADW1 = 33
HG = 5
PADW2 = 129


def kernel(x, input_embedding_table):
    B, H = x.shape
    V, D = input_embedding_table.shape
    n = B * H
    per_w = n // NW
    scale = float(np.sqrt(np.float32(D)))

    tcol_full = V // 128
    tail_w = V - tcol_full * 128
    main = (tcol_full // NW) // NBUF_T * NBUF_T

    PB = 128
    BLK = PB * H
    n_blk = per_w // BLK
    PS_DT = 8 * PADW2
    assert D == 2 * LANES and B % (NW * PB) == 0 and H % (2 * HG) == 0

    table_t = input_embedding_table.T
    idx = x.reshape(n)
    mesh = plsc.VectorSubcoreMesh(core_axis_name="c", subcore_axis_name="s")

    @functools.partial(
        pl.kernel,
        mesh=mesh,
        compiler_params=dataclasses.replace(
            pltpu.CompilerParams(use_tc_tiling_on_sc=True),
            needs_layout_passes=False,
        ),
        out_type=jax.ShapeDtypeStruct((V * D,), jnp.float32),
        scratch_types=(
            [pltpu.VMEM((4, 8, 128), jnp.float32)] * NBUF_T
            + [pltpu.VMEM((128 * 32,), jnp.float32)] * NBUF_T
            + [
                pltpu.VMEM((128 * 33,), jnp.float32),
                pltpu.VMEM((4, 8, tail_w), jnp.float32),
                pltpu.VMEM((tail_w * 32,), jnp.float32),
                pltpu.SemaphoreType.DMA((NBUF_T,)),
                pltpu.SemaphoreType.DMA((NBUF_T,)),
            ]
        ),
    )
    def sc_transpose(tbl_hbm, out_hbm, *scr):
        in_bufs = scr[0:NBUF_T]
        out_bufs = scr[NBUF_T : 2 * NBUF_T]
        pad_v, tin_v, tout_v, isem, osem = scr[2 * NBUF_T :]
        wid = lax.axis_index("s") * NC + lax.axis_index("c")
        iota = lax.iota(jnp.int32, LANES)

        def col_in_start(c, b):
            for dt in range(4):
                pltpu.async_copy(
                    tbl_hbm.at[pl.ds(dt * 8, 8), pl.ds(c * 128, 128)],
                    in_bufs[b].at[dt],
                    isem.at[b],
                )

        def col_in_wait(b):
            for dt in range(4):
                pltpu.make_async_copy(
                    tbl_hbm.at[pl.ds(dt * 8, 8), pl.ds(0, 128)],
                    in_bufs[b].at[dt],
                    isem.at[b],
                ).wait()

        def out_wait(b):
            pltpu.make_async_copy(
                out_bufs[b],
                out_hbm.at[pl.ds(0, 128 * 32)],
                osem.at[b],
            ).wait()

        iota33 = iota * 33

        def transpose_into(inref, outref, width, pad_buf):
            @plsc.parallel_loop(0, width // 16, unroll=1)
            def _(l0):
                for d in range(32):
                    dt, sl = d // 8, d % 8
                    v = inref[dt, sl, pl.ds(l0 * 16, 16)]
                    plsc.store_scatter(
                        pad_buf, [iota33 + (l0 * 528 + d)], v * scale
                    )

            @plsc.parallel_loop(0, width, unroll=4)
            def _(l):
                g0 = plsc.load_gather(pad_buf, [iota + l * 33])
                g1 = plsc.load_gather(pad_buf, [iota + (l * 33 + 16)])
                outref[pl.ds(l * 32, LANES)] = g0
                outref[pl.ds(l * 32 + LANES, LANES)] = g1

        for b0 in range(NBUF_T):
            col_in_start(wid + b0 * NW, b0)

        @pl.loop(0, main, step=NBUF_T)
        def _(j):
            for b in range(NBUF_T):
                jj = j + b
                c = wid + jj * NW
                col_in_wait(b)

                @pl.when(jj >= NBUF_T)
                def _():
                    out_wait(b)

                transpose_into(in_bufs[b], out_bufs[b], 128, pad_v)

                pltpu.async_copy(
                    out_bufs[b],
                    out_hbm.at[pl.ds(c * 4096, 4096)],
                    osem.at[b],
                )

                @pl.when(jj + NBUF_T < main)
                def _():
                    col_in_start(wid + (jj + NBUF_T) * NW, b)

        for b in range(NBUF_T):
            out_wait(b)

        nrest = tcol_full - main * NW
        rest_per_w = nrest // NW
        extra = nrest - rest_per_w * NW
        for r in range(rest_per_w):
            c = (main + r) * NW + wid
            for dt in range(4):
                pltpu.sync_copy(
                    tbl_hbm.at[pl.ds(dt * 8, 8), pl.ds(c * 128, 128)],
                    in_bufs[0].at[dt],
                )
            transpose_into(in_bufs[0], out_bufs[0], 128, pad_v)
            pltpu.sync_copy(out_bufs[0], out_hbm.at[pl.ds(c * 4096, 4096)])

        if extra:
            @pl.when(wid < extra)
            def _():
                c = (main + rest_per_w) * NW + wid
                for dt in range(4):
                    pltpu.sync_copy(
                        tbl_hbm.at[pl.ds(dt * 8, 8), pl.ds(c * 128, 128)],
                        in_bufs[0].at[dt],
                    )
                transpose_into(in_bufs[0], out_bufs[0], 128, pad_v)
                pltpu.sync_copy(
                    out_bufs[0], out_hbm.at[pl.ds(c * 4096, 4096)]
                )

        @pl.when(wid == NW - 1)
        def _():
            for dt in range(4):
                pltpu.sync_copy(
                    tbl_hbm.at[pl.ds(dt * 8, 8), pl.ds(tcol_full * 128, tail_w)],
                    tin_v.at[dt],
                )
            transpose_into(tin_v, tout_v, tail_w, pad_v)
            pltpu.sync_copy(
                tout_v, out_hbm.at[pl.ds(tcol_full * 4096, tail_w * 32)]
            )

    tbl_lin = sc_transpose(table_t)
    tbl2d = tbl_lin.reshape(V, D)

    @functools.partial(
        pl.kernel,
        mesh=mesh,
        compiler_params=dataclasses.replace(
            pltpu.CompilerParams(use_tc_tiling_on_sc=False),
            needs_layout_passes=False,
        ),
        out_type=jax.ShapeDtypeStruct(
            (H, D // 8, B // 128, 8, 128), jnp.float32
        ),
        scratch_types=[
            pltpu.VMEM((per_w,), jnp.int32),
            pltpu.VMEM((BLK, D), jnp.float32),
            pltpu.VMEM((HG * D, PADW2), jnp.float32),
            pltpu.SemaphoreType.DMA,
            pltpu.SemaphoreType.DMA,
        ],
    )
    def sc_gather(table_hbm, idx_hbm, out_hbm, idx_v, rows_v, piece_v, gsem, osem):
        wid = lax.axis_index("s") * NC + lax.axis_index("c")
        base = wid * per_w
        iota = lax.iota(jnp.int32, LANES)
        rowv = [
            [iota + (hl * D + g * LANES) for g in range(2)] for hl in range(HG)
        ]

        pltpu.sync_copy(idx_hbm.at[pl.ds(base, per_w)], idx_v)

        for blk in range(n_blk):
            pltpu.async_copy(
                table_hbm.at[idx_v.at[pl.ds(blk * BLK, BLK)]],
                rows_v,
                gsem,
            ).wait()

            @pl.loop(0, H // HG)
            def _(hg):
                h0 = hg * HG

                @plsc.parallel_loop(0, PB, unroll=2)
                def _(bi):
                    colv = jnp.full((LANES,), 0, jnp.int32) + bi
                    rb = bi * H + h0
                    for hl in range(HG):
                        r = rb + hl
                        v0 = rows_v[r, pl.ds(0, LANES)]
                        v1 = rows_v[r, pl.ds(LANES, LANES)]
                        plsc.store_scatter(piece_v, [rowv[hl][0], colv], v0)
                        plsc.store_scatter(piece_v, [rowv[hl][1], colv], v1)

                @pl.loop(0, HG * 4)
                def _(pc):
                    hl = pc // 4
                    dt = pc % 4
                    pltpu.async_copy(
                        piece_v.at[pl.ds(hl * D + dt * 8, 8), pl.ds(0, 128)],
                        out_hbm.at[h0 + hl, dt, wid * n_blk + blk],
                        osem,
                    )

                @pl.loop(0, HG * 4)
                def _(pc):
                    pltpu.make_async_copy(
                        piece_v.at[pl.ds(0, 8), pl.ds(0, 128)],
                        out_hbm.at[0, 0, 0],
                        osem,
                    ).wait()

    out5 = sc_gather(tbl2d, idx)
    return out5.transpose(2, 4, 0, 1, 3).reshape(B, H, D)


# speedup vs baseline: 3.0594x; 1.3983x over previous
import dataclasses
import functools

import numpy as np
import jax
import jax.numpy as jnp
from jax import lax
from jax.experimental import pallas as pl
from jax.experimental.pallas import tpu as pltpu
from jax.experimental.pallas import tpu_sc as plsc

NC = 2
NS = 16
NW = NC * NS
LANES = 16
NBUF_T = 8
PADW1 = 33
HG = 5
PADW2 = 129


def kernel(x, input_embedding_table):
    B, H = x.shape
    V, D = input_embedding_table.shape
    n = B * H
    per_w = n // NW
    scale = float(np.sqrt(np.float32(D)))

    tcol_full = V // 128
    tail_w = V - tcol_full * 128
    main = (tcol_full // NW) // NBUF_T * NBUF_T

    PB = 128
    BLK = PB * H
    n_blk = per_w // BLK
    PS_DT = 8 * PADW2
    assert D == 2 * LANES and B % (NW * PB) == 0 and H % (2 * HG) == 0

    table_t = input_embedding_table.T
    idx = x.reshape(n)
    mesh = plsc.VectorSubcoreMesh(core_axis_name="c", subcore_axis_name="s")

    @functools.partial(
        pl.kernel,
        mesh=mesh,
        compiler_params=dataclasses.replace(
            pltpu.CompilerParams(use_tc_tiling_on_sc=True),
            needs_layout_passes=False,
        ),
        out_type=jax.ShapeDtypeStruct((V * D,), jnp.float32),
        scratch_types=(
            [pltpu.VMEM((4, 8, 128), jnp.float32)] * NBUF_T
            + [pltpu.VMEM((128 * 32,), jnp.float32)] * NBUF_T
            + [
                pltpu.VMEM((128 * 33,), jnp.float32),
                pltpu.VMEM((4, 8, tail_w), jnp.float32),
                pltpu.VMEM((tail_w * 32,), jnp.float32),
                pltpu.SemaphoreType.DMA((NBUF_T,)),
                pltpu.SemaphoreType.DMA((NBUF_T,)),
            ]
        ),
    )
    def sc_transpose(tbl_hbm, out_hbm, *scr):
        in_bufs = scr[0:NBUF_T]
        out_bufs = scr[NBUF_T : 2 * NBUF_T]
        pad_v, tin_v, tout_v, isem, osem = scr[2 * NBUF_T :]
        wid = lax.axis_index("s") * NC + lax.axis_index("c")
        iota = lax.iota(jnp.int32, LANES)

        def col_in_start(c, b):
            for dt in range(4):
                pltpu.async_copy(
                    tbl_hbm.at[pl.ds(dt * 8, 8), pl.ds(c * 128, 128)],
                    in_bufs[b].at[dt],
                    isem.at[b],
                )

        def col_in_wait(b):
            for dt in range(4):
                pltpu.make_async_copy(
                    tbl_hbm.at[pl.ds(dt * 8, 8), pl.ds(0, 128)],
                    in_bufs[b].at[dt],
                    isem.at[b],
                ).wait()

        def out_wait(b):
            pltpu.make_async_copy(
                out_bufs[b],
                out_hbm.at[pl.ds(0, 128 * 32)],
                osem.at[b],
            ).wait()

        iota33 = iota * 33

        def transpose_into(inref, outref, width, pad_buf):
            @plsc.parallel_loop(0, width // 16, unroll=1)
            def _(l0):
                for d in range(32):
                    dt, sl = d // 8, d % 8
                    v = inref[dt, sl, pl.ds(l0 * 16, 16)]
                    plsc.store_scatter(
                        pad_buf, [iota33 + (l0 * 528 + d)], v
                    )

            @plsc.parallel_loop(0, width, unroll=4)
            def _(l):
                outref[pl.ds(l * 32, LANES)] = pad_buf[pl.ds(l * 33, LANES)]
                outref[pl.ds(l * 32 + LANES, LANES)] = pad_buf[
                    pl.ds(l * 33 + LANES, LANES)
                ]

        for b0 in range(NBUF_T):
            col_in_start(wid + b0 * NW, b0)

        @pl.loop(0, main, step=NBUF_T)
        def _(j):
            for b in range(NBUF_T):
                jj = j + b
                c = wid + jj * NW
                col_in_wait(b)

                @pl.when(jj >= NBUF_T)
                def _():
                    out_wait(b)

                transpose_into(in_bufs[b], out_bufs[b], 128, pad_v)

                pltpu.async_copy(
                    out_bufs[b],
                    out_hbm.at[pl.ds(c * 4096, 4096)],
                    osem.at[b],
                )

                @pl.when(jj + NBUF_T < main)
                def _():
                    col_in_start(wid + (jj + NBUF_T) * NW, b)

        for b in range(NBUF_T):
            out_wait(b)

        nrest = tcol_full - main * NW
        rest_per_w = nrest // NW
        extra = nrest - rest_per_w * NW
        for r in range(rest_per_w):
            c = (main + r) * NW + wid
            for dt in range(4):
                pltpu.sync_copy(
                    tbl_hbm.at[pl.ds(dt * 8, 8), pl.ds(c * 128, 128)],
                    in_bufs[0].at[dt],
                )
            transpose_into(in_bufs[0], out_bufs[0], 128, pad_v)
            pltpu.sync_copy(out_bufs[0], out_hbm.at[pl.ds(c * 4096, 4096)])

        if extra:
            @pl.when(wid < extra)
            def _():
                c = (main + rest_per_w) * NW + wid
                for dt in range(4):
                    pltpu.sync_copy(
                        tbl_hbm.at[pl.ds(dt * 8, 8), pl.ds(c * 128, 128)],
                        in_bufs[0].at[dt],
                    )
                transpose_into(in_bufs[0], out_bufs[0], 128, pad_v)
                pltpu.sync_copy(
                    out_bufs[0], out_hbm.at[pl.ds(c * 4096, 4096)]
                )

        @pl.when(wid == NW - 1)
        def _():
            for dt in range(4):
                pltpu.sync_copy(
                    tbl_hbm.at[pl.ds(dt * 8, 8), pl.ds(tcol_full * 128, tail_w)],
                    tin_v.at[dt],
                )
            transpose_into(tin_v, tout_v, tail_w, pad_v)
            pltpu.sync_copy(
                tout_v, out_hbm.at[pl.ds(tcol_full * 4096, tail_w * 32)]
            )

    tbl_lin = sc_transpose(table_t)
    tbl2d = tbl_lin.reshape(V, D)

    @functools.partial(
        pl.kernel,
        mesh=mesh,
        compiler_params=dataclasses.replace(
            pltpu.CompilerParams(use_tc_tiling_on_sc=False),
            needs_layout_passes=False,
        ),
        out_type=jax.ShapeDtypeStruct(
            (H, D // 8, B // 128, 8, 128), jnp.float32
        ),
        scratch_types=[
            pltpu.VMEM((per_w,), jnp.int32),
            pltpu.VMEM((BLK, D), jnp.float32),
            pltpu.VMEM((HG * D, PADW2), jnp.float32),
            pltpu.SemaphoreType.DMA,
            pltpu.SemaphoreType.DMA,
        ],
    )
    def sc_gather(table_hbm, idx_hbm, out_hbm, idx_v, rows_v, piece_v, gsem, osem):
        wid = lax.axis_index("s") * NC + lax.axis_index("c")
        base = wid * per_w
        iota = lax.iota(jnp.int32, LANES)
        rowv = [
            [iota + (hl * D + g * LANES) for g in range(2)] for hl in range(HG)
        ]

        pltpu.sync_copy(idx_hbm.at[pl.ds(base, per_w)], idx_v)

        for blk in range(n_blk):
            pltpu.async_copy(
                table_hbm.at[idx_v.at[pl.ds(blk * BLK, BLK)]],
                rows_v,
                gsem,
            ).wait()

            @pl.loop(0, H // HG)
            def _(hg):
                h0 = hg * HG

                @plsc.parallel_loop(0, PB, unroll=2)
                def _(bi):
                    colv = jnp.full((LANES,), 0, jnp.int32) + bi
                    rb = bi * H + h0
                    for hl in range(HG):
                        r = rb + hl
                        v0 = rows_v[r, pl.ds(0, LANES)] * scale
                        v1 = rows_v[r, pl.ds(LANES, LANES)] * scale
                        plsc.store_scatter(piece_v, [rowv[hl][0], colv], v0)
                        plsc.store_scatter(piece_v, [rowv[hl][1], colv], v1)

                @pl.loop(0, HG * 4)
                def _(pc):
                    hl = pc // 4
                    dt = pc % 4
                    pltpu.async_copy(
                        piece_v.at[pl.ds(hl * D + dt * 8, 8), pl.ds(0, 128)],
                        out_hbm.at[h0 + hl, dt, wid * n_blk + blk],
                        osem,
                    )

                @pl.loop(0, HG * 4)
                def _(pc):
                    pltpu.make_async_copy(
                        piece_v.at[pl.ds(0, 8), pl.ds(0, 128)],
                        out_hbm.at[0, 0, 0],
                        osem,
                    ).wait()

    out5 = sc_gather(tbl2d, idx)
    return out5.transpose(2, 4, 0, 1, 3).reshape(B, H, D)


# speedup vs baseline: 3.1454x; 1.0281x over previous
import dataclasses
import functools

import numpy as np
import jax
import jax.numpy as jnp
from jax import lax
from jax.experimental import pallas as pl
from jax.experimental.pallas import tpu as pltpu
from jax.experimental.pallas import tpu_sc as plsc

NC = 2
NS = 16
NW = NC * NS
LANES = 16
NBUF_T = 8
PADW1 = 33
HG = 5
PADW2 = 129


def kernel(x, input_embedding_table):
    B, H = x.shape
    V, D = input_embedding_table.shape
    n = B * H
    per_w = n // NW
    scale = float(np.sqrt(np.float32(D)))

    tcol_full = V // 128
    tail_w = V - tcol_full * 128
    main = (tcol_full // NW) // NBUF_T * NBUF_T

    PB = 64
    BLK = PB * H
    n_blk = per_w // BLK
    PADW2 = 65
    assert D == 2 * LANES and B % (NW * 2 * PB) == 0 and H % (2 * HG) == 0

    table_t = input_embedding_table.T
    idx = x.reshape(n)
    mesh = plsc.VectorSubcoreMesh(core_axis_name="c", subcore_axis_name="s")

    @functools.partial(
        pl.kernel,
        mesh=mesh,
        compiler_params=dataclasses.replace(
            pltpu.CompilerParams(use_tc_tiling_on_sc=True),
            needs_layout_passes=False,
        ),
        out_type=jax.ShapeDtypeStruct((V * D,), jnp.float32),
        scratch_types=(
            [pltpu.VMEM((4, 8, 128), jnp.float32)] * NBUF_T
            + [pltpu.VMEM((128 * 32,), jnp.float32)] * NBUF_T
            + [
                pltpu.VMEM((128 * 33,), jnp.float32),
                pltpu.VMEM((4, 8, tail_w), jnp.float32),
                pltpu.VMEM((tail_w * 32,), jnp.float32),
                pltpu.SemaphoreType.DMA((NBUF_T,)),
                pltpu.SemaphoreType.DMA((NBUF_T,)),
            ]
        ),
    )
    def sc_transpose(tbl_hbm, out_hbm, *scr):
        in_bufs = scr[0:NBUF_T]
        out_bufs = scr[NBUF_T : 2 * NBUF_T]
        pad_v, tin_v, tout_v, isem, osem = scr[2 * NBUF_T :]
        wid = lax.axis_index("s") * NC + lax.axis_index("c")
        iota = lax.iota(jnp.int32, LANES)

        def col_in_start(c, b):
            for dt in range(4):
                pltpu.async_copy(
                    tbl_hbm.at[pl.ds(dt * 8, 8), pl.ds(c * 128, 128)],
                    in_bufs[b].at[dt],
                    isem.at[b],
                )

        def col_in_wait(b):
            for dt in range(4):
                pltpu.make_async_copy(
                    tbl_hbm.at[pl.ds(dt * 8, 8), pl.ds(0, 128)],
                    in_bufs[b].at[dt],
                    isem.at[b],
                ).wait()

        def out_wait(b):
            pltpu.make_async_copy(
                out_bufs[b],
                out_hbm.at[pl.ds(0, 128 * 32)],
                osem.at[b],
            ).wait()

        iota33 = iota * 33

        def transpose_into(inref, outref, width, pad_buf):
            @plsc.parallel_loop(0, width // 16, unroll=1)
            def _(l0):
                for d in range(32):
                    dt, sl = d // 8, d % 8
                    v = inref[dt, sl, pl.ds(l0 * 16, 16)]
                    plsc.store_scatter(
                        pad_buf, [iota33 + (l0 * 528 + d)], v
                    )

            @plsc.parallel_loop(0, width, unroll=4)
            def _(l):
                outref[pl.ds(l * 32, LANES)] = pad_buf[pl.ds(l * 33, LANES)]
                outref[pl.ds(l * 32 + LANES, LANES)] = pad_buf[
                    pl.ds(l * 33 + LANES, LANES)
                ]

        for b0 in range(NBUF_T):
            col_in_start(wid + b0 * NW, b0)

        @pl.loop(0, main, step=NBUF_T)
        def _(j):
            for b in range(NBUF_T):
                jj = j + b
                c = wid + jj * NW
                col_in_wait(b)

                @pl.when(jj >= NBUF_T)
                def _():
                    out_wait(b)

                transpose_into(in_bufs[b], out_bufs[b], 128, pad_v)

                pltpu.async_copy(
                    out_bufs[b],
                    out_hbm.at[pl.ds(c * 4096, 4096)],
                    osem.at[b],
                )

                @pl.when(jj + NBUF_T < main)
                def _():
                    col_in_start(wid + (jj + NBUF_T) * NW, b)

        for b in range(NBUF_T):
            out_wait(b)

        nrest = tcol_full - main * NW
        rest_per_w = nrest // NW
        extra = nrest - rest_per_w * NW
        for r in range(rest_per_w):
            c = (main + r) * NW + wid
            for dt in range(4):
                pltpu.sync_copy(
                    tbl_hbm.at[pl.ds(dt * 8, 8), pl.ds(c * 128, 128)],
                    in_bufs[0].at[dt],
                )
            transpose_into(in_bufs[0], out_bufs[0], 128, pad_v)
            pltpu.sync_copy(out_bufs[0], out_hbm.at[pl.ds(c * 4096, 4096)])

        if extra:
            @pl.when(wid < extra)
            def _():
                c = (main + rest_per_w) * NW + wid
                for dt in range(4):
                    pltpu.sync_copy(
                        tbl_hbm.at[pl.ds(dt * 8, 8), pl.ds(c * 128, 128)],
                        in_bufs[0].at[dt],
                    )
                transpose_into(in_bufs[0], out_bufs[0], 128, pad_v)
                pltpu.sync_copy(
                    out_bufs[0], out_hbm.at[pl.ds(c * 4096, 4096)]
                )

        @pl.when(wid == NW - 1)
        def _():
            for dt in range(4):
                pltpu.sync_copy(
                    tbl_hbm.at[pl.ds(dt * 8, 8), pl.ds(tcol_full * 128, tail_w)],
                    tin_v.at[dt],
                )
            transpose_into(tin_v, tout_v, tail_w, pad_v)
            pltpu.sync_copy(
                tout_v, out_hbm.at[pl.ds(tcol_full * 4096, tail_w * 32)]
            )

    tbl_lin = sc_transpose(table_t)
    tbl2d = tbl_lin.reshape(V, D)

    @functools.partial(
        pl.kernel,
        mesh=mesh,
        compiler_params=dataclasses.replace(
            pltpu.CompilerParams(use_tc_tiling_on_sc=False),
            needs_layout_passes=False,
        ),
        out_type=jax.ShapeDtypeStruct(
            (H, D // 8, B // 128, 8, 128), jnp.float32
        ),
        scratch_types=[
            pltpu.VMEM((per_w,), jnp.int32),
            pltpu.VMEM((BLK, D), jnp.float32),
            pltpu.VMEM((BLK, D), jnp.float32),
            pltpu.VMEM((HG * D, PADW2), jnp.float32),
            pltpu.SemaphoreType.DMA((2,)),
            pltpu.SemaphoreType.DMA,
        ],
    )
    def sc_gather(
        table_hbm, idx_hbm, out_hbm, idx_v, rows0_v, rows1_v, piece_v, gsem, osem
    ):
        wid = lax.axis_index("s") * NC + lax.axis_index("c")
        base = wid * per_w
        iota = lax.iota(jnp.int32, LANES)
        rows_bufs = (rows0_v, rows1_v)
        rowv = [
            [iota + (hl * D + g * LANES) for g in range(2)] for hl in range(HG)
        ]

        pltpu.sync_copy(idx_hbm.at[pl.ds(base, per_w)], idx_v)

        def start_gather(u):
            return pltpu.async_copy(
                table_hbm.at[idx_v.at[pl.ds(u * BLK, BLK)]],
                rows_bufs[u % 2],
                gsem.at[u % 2],
            )

        gat = {0: start_gather(0), 1: start_gather(1)}
        for u in range(n_blk):
            rows_v = rows_bufs[u % 2]
            btg = wid * (n_blk // 2) + u // 2
            bh = u % 2
            gat[u].wait()

            @pl.loop(0, H // HG)
            def _(hg):
                h0 = hg * HG

                @plsc.parallel_loop(0, PB, unroll=2)
                def _(bi):
                    colv = jnp.full((LANES,), 0, jnp.int32) + bi
                    rb = bi * H + h0
                    for hl in range(HG):
                        r = rb + hl
                        v0 = rows_v[r, pl.ds(0, LANES)] * scale
                        v1 = rows_v[r, pl.ds(LANES, LANES)] * scale
                        plsc.store_scatter(piece_v, [rowv[hl][0], colv], v0)
                        plsc.store_scatter(piece_v, [rowv[hl][1], colv], v1)

                @pl.loop(0, HG * 4)
                def _(pc):
                    hl = pc // 4
                    dt = pc % 4
                    pltpu.async_copy(
                        piece_v.at[pl.ds(hl * D + dt * 8, 8), pl.ds(0, PB)],
                        out_hbm.at[h0 + hl, dt, btg, :, pl.ds(bh * PB, PB)],
                        osem,
                    )

                @pl.loop(0, HG * 4)
                def _(pc):
                    pltpu.make_async_copy(
                        piece_v.at[pl.ds(0, 8), pl.ds(0, PB)],
                        out_hbm.at[0, 0, 0, :, pl.ds(0, PB)],
                        osem,
                    ).wait()

            if u + 2 < n_blk:
                gat[u + 2] = start_gather(u + 2)

    out5 = sc_gather(tbl2d, idx)
    return out5.transpose(2, 4, 0, 1, 3).reshape(B, H, D)


# speedup vs baseline: 3.2081x; 1.0199x over previous
import dataclasses
import functools

import numpy as np
import jax
import jax.numpy as jnp
from jax import lax
from jax.experimental import pallas as pl
from jax.experimental.pallas import tpu as pltpu
from jax.experimental.pallas import tpu_sc as plsc

NC = 2
NS = 16
NW = NC * NS
LANES = 16
NBUF_T = 8
PADW1 = 33
HG = 5
PADW2 = 129


def kernel(x, input_embedding_table):
    B, H = x.shape
    V, D = input_embedding_table.shape
    n = B * H
    per_w = n // NW
    scale = float(np.sqrt(np.float32(D)))

    tcol_full = V // 128
    tail_w = V - tcol_full * 128
    main = (tcol_full // NW) // NBUF_T * NBUF_T

    PB = 64
    BLK = PB * H
    n_blk = per_w // BLK
    PADW2 = 65
    assert D == 2 * LANES and B % (NW * 2 * PB) == 0 and H % (2 * HG) == 0

    table_t = input_embedding_table.T
    idx = x.reshape(n)
    mesh = plsc.VectorSubcoreMesh(core_axis_name="c", subcore_axis_name="s")

    @functools.partial(
        pl.kernel,
        mesh=mesh,
        compiler_params=dataclasses.replace(
            pltpu.CompilerParams(use_tc_tiling_on_sc=True),
            needs_layout_passes=False,
        ),
        out_type=jax.ShapeDtypeStruct((V * D,), jnp.float32),
        scratch_types=(
            [pltpu.VMEM((32, 128), jnp.float32)] * NBUF_T
            + [pltpu.VMEM((128 * 32,), jnp.float32)] * NBUF_T
            + [
                pltpu.VMEM((128 * 33,), jnp.float32),
                pltpu.VMEM((32, tail_w), jnp.float32),
                pltpu.VMEM((tail_w * 32,), jnp.float32),
                pltpu.SemaphoreType.DMA((NBUF_T,)),
                pltpu.SemaphoreType.DMA((NBUF_T,)),
            ]
        ),
    )
    def sc_transpose(tbl_hbm, out_hbm, *scr):
        in_bufs = scr[0:NBUF_T]
        out_bufs = scr[NBUF_T : 2 * NBUF_T]
        pad_v, tin_v, tout_v, isem, osem = scr[2 * NBUF_T :]
        wid = lax.axis_index("s") * NC + lax.axis_index("c")
        iota = lax.iota(jnp.int32, LANES)

        def col_in_start(c, b):
            pltpu.async_copy(
                tbl_hbm.at[:, pl.ds(c * 128, 128)],
                in_bufs[b],
                isem.at[b],
            )

        def col_in_wait(b):
            pltpu.make_async_copy(
                tbl_hbm.at[:, pl.ds(0, 128)],
                in_bufs[b],
                isem.at[b],
            ).wait()

        def out_wait(b):
            pltpu.make_async_copy(
                out_bufs[b],
                out_hbm.at[pl.ds(0, 128 * 32)],
                osem.at[b],
            ).wait()

        iota33 = iota * 33

        def transpose_into(inref, outref, width, pad_buf):
            @plsc.parallel_loop(0, width // 16, unroll=2)
            def _(l0):
                for d in range(32):
                    v = inref[d, pl.ds(l0 * 16, 16)]
                    plsc.store_scatter(
                        pad_buf, [iota33 + (l0 * 528 + d)], v
                    )

            @plsc.parallel_loop(0, width, unroll=8)
            def _(l):
                outref[pl.ds(l * 32, LANES)] = pad_buf[pl.ds(l * 33, LANES)]
                outref[pl.ds(l * 32 + LANES, LANES)] = pad_buf[
                    pl.ds(l * 33 + LANES, LANES)
                ]

        for b0 in range(NBUF_T):
            col_in_start(wid + b0 * NW, b0)

        @pl.loop(0, main, step=NBUF_T)
        def _(j):
            for b in range(NBUF_T):
                jj = j + b
                c = wid + jj * NW
                col_in_wait(b)

                @pl.when(jj >= NBUF_T)
                def _():
                    out_wait(b)

                transpose_into(in_bufs[b], out_bufs[b], 128, pad_v)

                pltpu.async_copy(
                    out_bufs[b],
                    out_hbm.at[pl.ds(c * 4096, 4096)],
                    osem.at[b],
                )

                @pl.when(jj + NBUF_T < main)
                def _():
                    col_in_start(wid + (jj + NBUF_T) * NW, b)

        for b in range(NBUF_T):
            out_wait(b)

        nrest = tcol_full - main * NW
        rest_per_w = nrest // NW
        extra = nrest - rest_per_w * NW
        for r in range(rest_per_w):
            c = (main + r) * NW + wid
            pltpu.sync_copy(
                tbl_hbm.at[:, pl.ds(c * 128, 128)], in_bufs[0]
            )
            transpose_into(in_bufs[0], out_bufs[0], 128, pad_v)
            pltpu.sync_copy(out_bufs[0], out_hbm.at[pl.ds(c * 4096, 4096)])

        if extra:
            @pl.when(wid < extra)
            def _():
                c = (main + rest_per_w) * NW + wid
                pltpu.sync_copy(
                    tbl_hbm.at[:, pl.ds(c * 128, 128)], in_bufs[0]
                )
                transpose_into(in_bufs[0], out_bufs[0], 128, pad_v)
                pltpu.sync_copy(
                    out_bufs[0], out_hbm.at[pl.ds(c * 4096, 4096)]
                )

        @pl.when(wid == NW - 1)
        def _():
            pltpu.sync_copy(
                tbl_hbm.at[:, pl.ds(tcol_full * 128, tail_w)], tin_v
            )
            transpose_into(tin_v, tout_v, tail_w, pad_v)
            pltpu.sync_copy(
                tout_v, out_hbm.at[pl.ds(tcol_full * 4096, tail_w * 32)]
            )

    tbl_lin = sc_transpose(table_t)
    tbl2d = tbl_lin.reshape(V, D)

    @functools.partial(
        pl.kernel,
        mesh=mesh,
        compiler_params=dataclasses.replace(
            pltpu.CompilerParams(use_tc_tiling_on_sc=False),
            needs_layout_passes=False,
        ),
        out_type=jax.ShapeDtypeStruct(
            (H, D // 8, B // 128, 8, 128), jnp.float32
        ),
        scratch_types=[
            pltpu.VMEM((per_w,), jnp.int32),
            pltpu.VMEM((BLK, D), jnp.float32),
            pltpu.VMEM((BLK, D), jnp.float32),
            pltpu.VMEM((HG * D, PADW2), jnp.float32),
            pltpu.SemaphoreType.DMA((2,)),
            pltpu.SemaphoreType.DMA,
        ],
    )
    def sc_gather(
        table_hbm, idx_hbm, out_hbm, idx_v, rows0_v, rows1_v, piece_v, gsem, osem
    ):
        wid = lax.axis_index("s") * NC + lax.axis_index("c")
        base = wid * per_w
        iota = lax.iota(jnp.int32, LANES)
        rows_bufs = (rows0_v, rows1_v)
        rowv = [
            [iota + (hl * D + g * LANES) for g in range(2)] for hl in range(HG)
        ]

        pltpu.sync_copy(idx_hbm.at[pl.ds(base, per_w)], idx_v)

        def start_gather(u):
            return pltpu.async_copy(
                table_hbm.at[idx_v.at[pl.ds(u * BLK, BLK)]],
                rows_bufs[u % 2],
                gsem.at[u % 2],
            )

        gat = {0: start_gather(0), 1: start_gather(1)}
        for u in range(n_blk):
            rows_v = rows_bufs[u % 2]
            btg = wid * (n_blk // 2) + u // 2
            bh = u % 2
            gat[u].wait()

            @pl.loop(0, H // HG)
            def _(hg):
                h0 = hg * HG

                @plsc.parallel_loop(0, PB, unroll=2)
                def _(bi):
                    colv = jnp.full((LANES,), 0, jnp.int32) + bi
                    rb = bi * H + h0
                    for hl in range(HG):
                        r = rb + hl
                        v0 = rows_v[r, pl.ds(0, LANES)] * scale
                        v1 = rows_v[r, pl.ds(LANES, LANES)] * scale
                        plsc.store_scatter(piece_v, [rowv[hl][0], colv], v0)
                        plsc.store_scatter(piece_v, [rowv[hl][1], colv], v1)

                @pl.loop(0, HG * 4)
                def _(pc):
                    hl = pc // 4
                    dt = pc % 4
                    pltpu.async_copy(
                        piece_v.at[pl.ds(hl * D + dt * 8, 8), pl.ds(0, PB)],
                        out_hbm.at[h0 + hl, dt, btg, :, pl.ds(bh * PB, PB)],
                        osem,
                    )

                @pl.loop(0, HG * 4)
                def _(pc):
                    pltpu.make_async_copy(
                        piece_v.at[pl.ds(0, 8), pl.ds(0, PB)],
                        out_hbm.at[0, 0, 0, :, pl.ds(0, PB)],
                        osem,
                    ).wait()

            if u + 2 < n_blk:
                gat[u + 2] = start_gather(u + 2)

    out5 = sc_gather(tbl2d, idx)
    return out5.transpose(2, 4, 0, 1, 3).reshape(B, H, D)
